# baseline (speedup 1.0000x reference)
import functools

import jax
import jax.numpy as jnp
from jax import lax
from jax.experimental import pallas as pl
from jax.experimental.pallas import tpu as pltpu
from jax.experimental.pallas import tpu_sc as plsc

N_ROWS = 100000
D = 128
S = 256
NC = 2
NS = 16
NW = NC * NS
CH = 128
FULL = (N_ROWS // (NW * CH)) * NW * CH
CHUNKS = FULL // (NW * CH)
EXTRA = (N_ROWS - FULL) // CH
TAIL = N_ROWS - FULL - EXTRA * CH
TAIL_OFF = FULL + EXTRA * CH


BIG = 384
NBIG = CHUNKS * CH // BIG


def _sc_body(feat_hbm, seg_hbm, psums_hbm, pcnts_hbm, out_hbm,
             acc, rows0, rows1, idx_all, idx_e, idx32, cnt, zbuf,
             cvm, pa, pb, obuf, inv,
             sem_g0, sem_g1, sem_s, sem_i, sem_b):
  c = lax.axis_index("c")
  s = lax.axis_index("s")
  w = s * NC + c
  base = w * (CHUNKS * CH)

  zv = jnp.zeros((16,), jnp.float32)
  ov = jnp.ones((16,), jnp.float32)
  for i in range(16):
    for k in range(D // 16):
      zbuf[i, pl.ds(k * 16, 16)] = zv
  for i in range(S // 16):
    cnt[pl.ds(i * 16, 16)] = zv

  sl = pl.ds(s * 16, 16)
  pltpu.sync_copy(zbuf, acc.at[sl])
  plsc.subcore_barrier()

  def count_ids(idx_vals):
    plsc.addupdate_scatter(cnt, [idx_vals], ov)

  bufs = (rows0, rows1)
  gsems = (sem_g0, sem_g1)

  idx_dmas = [
      pltpu.async_copy(seg_hbm.at[pl.ds(base + r * CH, CH)], idx_all.at[r],
                       sem_i)
      for r in range(CHUNKS)
  ]
  gathers = [pltpu.async_copy(feat_hbm.at[pl.ds(base, BIG)], bufs[0], sem_g0)]
  for d in idx_dmas:
    d.wait()

  pending_scatters = []
  for b in range(NBIG):
    for d in pending_scatters:
      d.wait()
    pending_scatters = []
    if b + 1 < NBIG:
      gathers.append(
          pltpu.async_copy(feat_hbm.at[pl.ds(base + (b + 1) * BIG, BIG)],
                           bufs[(b + 1) % 2], gsems[(b + 1) % 2]))
    gathers[b].wait()
    buf = bufs[b % 2]
    for k in range(BIG // CH):
      r = b * (BIG // CH) + k
      pending_scatters.append(
          pltpu.async_copy(buf.at[pl.ds(k * CH, CH)], acc.at[idx_all.at[r]],
                           sem_s, add=True))
    for k in range(BIG // CH):
      r = b * (BIG // CH) + k
      for q in range(CH // 16):
        count_ids(idx_all[r, pl.ds(q * 16, 16)])
  for d in pending_scatters:
    d.wait()

  @pl.when(w < EXTRA)
  def _():
    off = FULL + w * CH
    pltpu.sync_copy(seg_hbm.at[pl.ds(off, CH)], idx_e)
    pltpu.sync_copy(feat_hbm.at[pl.ds(off, CH)], rows0.at[pl.ds(0, CH)])
    pltpu.sync_copy(rows0.at[pl.ds(0, CH)], acc.at[idx_e], add=True)
    for q in range(CH // 16):
      count_ids(idx_e[pl.ds(q * 16, 16)])

  @pl.when(w == EXTRA)
  def _():
    pltpu.sync_copy(seg_hbm.at[pl.ds(TAIL_OFF, TAIL)], idx32)
    pltpu.sync_copy(feat_hbm.at[pl.ds(TAIL_OFF, TAIL)], rows1.at[pl.ds(0, TAIL)])
    pltpu.sync_copy(rows1.at[pl.ds(0, TAIL)], acc.at[idx32], add=True)
    for q in range(TAIL // 16):
      count_ids(idx32[pl.ds(q * 16, 16)])

  plsc.subcore_barrier()

  pltpu.sync_copy(acc.at[sl], psums_hbm.at[c, sl])
  pltpu.sync_copy(cnt, pcnts_hbm.at[w])

  plsc.subcore_barrier()
  pltpu.core_barrier(sem_b, core_axis_name="c")

  pltpu.sync_copy(pcnts_hbm, cvm)
  ro = pl.ds(w * 8, 8)
  pltpu.sync_copy(psums_hbm.at[0, ro], pa)
  pltpu.sync_copy(psums_hbm.at[1, ro], pb)
  one = jnp.ones((16,), jnp.float32)
  for i in range(S // 16):
    tot = jnp.zeros((16,), jnp.float32)
    for t in range(NW):
      tot = tot + cvm[t, pl.ds(i * 16, 16)]
    inv[pl.ds(i * 16, 16)] = one / jnp.maximum(tot, one)
  for r in range(8):
    x = plsc.load_gather(inv, [jnp.full((16,), w * 8 + r, jnp.int32)])
    for k in range(D // 16):
      co = pl.ds(k * 16, 16)
      obuf[r, co] = (pa[r, co] + pb[r, co]) * x
  pltpu.sync_copy(obuf, out_hbm.at[ro])


_sc_pool = functools.partial(
    pl.kernel,
    out_type=(
        jax.ShapeDtypeStruct((NC, S, D), jnp.float32),
        jax.ShapeDtypeStruct((NW, S), jnp.float32),
        jax.ShapeDtypeStruct((S, D), jnp.float32),
    ),
    mesh=plsc.VectorSubcoreMesh(
        core_axis_name="c", subcore_axis_name="s",
        num_cores=NC, num_subcores=NS),
    scratch_types=[
        pltpu.VMEM_SHARED((S, D), jnp.float32),
        pltpu.VMEM((BIG, D), jnp.float32),
        pltpu.VMEM((BIG, D), jnp.float32),
        pltpu.VMEM((CHUNKS, CH), jnp.int32),
        pltpu.VMEM((CH,), jnp.int32),
        pltpu.VMEM((TAIL,), jnp.int32),
        pltpu.VMEM((S,), jnp.float32),
        pltpu.VMEM((16, D), jnp.float32),
        pltpu.VMEM((NW, S), jnp.float32),
        pltpu.VMEM((8, D), jnp.float32),
        pltpu.VMEM((8, D), jnp.float32),
        pltpu.VMEM((8, D), jnp.float32),
        pltpu.VMEM((S,), jnp.float32),
        pltpu.SemaphoreType.DMA,
        pltpu.SemaphoreType.DMA,
        pltpu.SemaphoreType.DMA,
        pltpu.SemaphoreType.DMA,
        pltpu.SemaphoreType.REGULAR,
    ],
    compiler_params=pltpu.CompilerParams(needs_layout_passes=False),
)(_sc_body)


def kernel(feat, segment_ids, num_graphs):
  seg = segment_ids.astype(jnp.int32)
  _, _, out = _sc_pool(feat, seg)
  return out


# speedup vs baseline: 1.1148x; 1.1148x over previous
import functools

import jax
import jax.numpy as jnp
from jax import lax
from jax.experimental import pallas as pl
from jax.experimental.pallas import tpu as pltpu
from jax.experimental.pallas import tpu_sc as plsc

N_ROWS = 100000
D = 128
S = 256
NC = 2
NS = 16
NW = NC * NS
CH = 128
FULL = (N_ROWS // (NW * CH)) * NW * CH
CHUNKS = FULL // (NW * CH)
EXTRA = (N_ROWS - FULL) // CH
TAIL = N_ROWS - FULL - EXTRA * CH
TAIL_OFF = FULL + EXTRA * CH


BIG = 256
NBUF = 3
NBIG = CHUNKS * CH // BIG


def _sc_body(feat_hbm, seg_hbm, psums_hbm, pcnts_hbm,
             acc, rows0, rows1, rows2, idx_all, idx_e, idx32, cnt, zbuf,
             sem_g0, sem_g1, sem_g2, sem_s, sem_i):
  c = lax.axis_index("c")
  s = lax.axis_index("s")
  w = s * NC + c
  base = w * (CHUNKS * CH)

  zv = jnp.zeros((16,), jnp.float32)
  ov = jnp.ones((16,), jnp.float32)
  for i in range(16):
    for k in range(D // 16):
      zbuf[i, pl.ds(k * 16, 16)] = zv
  for i in range(S // 16):
    cnt[pl.ds(i * 16, 16)] = zv

  sl = pl.ds(s * 16, 16)
  pltpu.sync_copy(zbuf, acc.at[sl])
  plsc.subcore_barrier()

  def count_ids(idx_vals):
    plsc.addupdate_scatter(cnt, [idx_vals], ov)

  bufs = (rows0, rows1, rows2)
  gsems = (sem_g0, sem_g1, sem_g2)

  def gather(b):
    return pltpu.async_copy(feat_hbm.at[pl.ds(base + b * BIG, BIG)],
                            bufs[b % NBUF], gsems[b % NBUF])

  idx_dmas = [
      pltpu.async_copy(seg_hbm.at[pl.ds(base + r * CH, CH)], idx_all.at[r],
                       sem_i)
      for r in range(CHUNKS)
  ]
  gathers = [gather(0), gather(1)]
  for d in idx_dmas:
    d.wait()

  scatters = {}
  for b in range(NBIG):
    for d in scatters.pop(b - 1, ()):
      d.wait()
    if b + 2 < NBIG:
      gathers.append(gather(b + 2))
    gathers[b].wait()
    buf = bufs[b % NBUF]
    scatters[b] = [
        pltpu.async_copy(buf.at[pl.ds(k * CH, CH)],
                         acc.at[idx_all.at[b * (BIG // CH) + k]],
                         sem_s, add=True)
        for k in range(BIG // CH)
    ]
    for k in range(BIG // CH):
      r = b * (BIG // CH) + k
      for q in range(CH // 16):
        count_ids(idx_all[r, pl.ds(q * 16, 16)])
  for ds_ in scatters.values():
    for d in ds_:
      d.wait()

  @pl.when(w < EXTRA)
  def _():
    off = FULL + w * CH
    pltpu.sync_copy(seg_hbm.at[pl.ds(off, CH)], idx_e)
    pltpu.sync_copy(feat_hbm.at[pl.ds(off, CH)], rows0.at[pl.ds(0, CH)])
    pltpu.sync_copy(rows0.at[pl.ds(0, CH)], acc.at[idx_e], add=True)
    for q in range(CH // 16):
      count_ids(idx_e[pl.ds(q * 16, 16)])

  @pl.when(w == EXTRA)
  def _():
    pltpu.sync_copy(seg_hbm.at[pl.ds(TAIL_OFF, TAIL)], idx32)
    pltpu.sync_copy(feat_hbm.at[pl.ds(TAIL_OFF, TAIL)], rows1.at[pl.ds(0, TAIL)])
    pltpu.sync_copy(rows1.at[pl.ds(0, TAIL)], acc.at[idx32], add=True)
    for q in range(TAIL // 16):
      count_ids(idx32[pl.ds(q * 16, 16)])

  plsc.subcore_barrier()

  pltpu.sync_copy(acc.at[sl], psums_hbm.at[c, sl])
  pltpu.sync_copy(cnt, pcnts_hbm.at[w])


_sc_pool = functools.partial(
    pl.kernel,
    out_type=(
        jax.ShapeDtypeStruct((NC, S, D), jnp.float32),
        jax.ShapeDtypeStruct((NW, S), jnp.float32),
    ),
    mesh=plsc.VectorSubcoreMesh(
        core_axis_name="c", subcore_axis_name="s",
        num_cores=NC, num_subcores=NS),
    scratch_types=[
        pltpu.VMEM_SHARED((S, D), jnp.float32),
        pltpu.VMEM((BIG, D), jnp.float32),
        pltpu.VMEM((BIG, D), jnp.float32),
        pltpu.VMEM((BIG, D), jnp.float32),
        pltpu.VMEM((CHUNKS, CH), jnp.int32),
        pltpu.VMEM((CH,), jnp.int32),
        pltpu.VMEM((TAIL,), jnp.int32),
        pltpu.VMEM((S,), jnp.float32),
        pltpu.VMEM((16, D), jnp.float32),
        pltpu.SemaphoreType.DMA,
        pltpu.SemaphoreType.DMA,
        pltpu.SemaphoreType.DMA,
        pltpu.SemaphoreType.DMA,
        pltpu.SemaphoreType.DMA,
    ],
    compiler_params=pltpu.CompilerParams(needs_layout_passes=False),
)(_sc_body)


def _combine_body(ps_ref, pc_ref, o_ref):
  sums = ps_ref[0] + ps_ref[1]
  counts = jnp.sum(pc_ref[...], axis=0, keepdims=True)
  counts_col = jnp.transpose(counts)
  o_ref[...] = sums / jnp.clip(counts_col, 1.0, None)


def kernel(feat, segment_ids, num_graphs):
  seg = segment_ids.astype(jnp.int32)
  psums, pcnts = _sc_pool(feat, seg)
  out = pl.pallas_call(
      _combine_body,
      out_shape=jax.ShapeDtypeStruct((S, D), jnp.float32),
  )(psums, pcnts)
  return out


# speedup vs baseline: 1.1159x; 1.0010x over previous
import functools

import jax
import jax.numpy as jnp
from jax import lax
from jax.experimental import pallas as pl
from jax.experimental.pallas import tpu as pltpu
from jax.experimental.pallas import tpu_sc as plsc

N_ROWS = 100000
D = 128
S = 256
NC = 2
NS = 16
NW = NC * NS
CH = 128
FULL = (N_ROWS // (NW * CH)) * NW * CH
CHUNKS = FULL // (NW * CH)
EXTRA = (N_ROWS - FULL) // CH
TAIL = N_ROWS - FULL - EXTRA * CH
TAIL_OFF = FULL + EXTRA * CH


BIG = 256
NBUF = 3
NBIG = CHUNKS * CH // BIG


def _sc_body(feat_hbm, seg_hbm, psums_hbm, pcnts_hbm,
             acc, rows0, rows1, rows2, idx_all, idx_e, idx32, cnt, zbuf,
             sem_g0, sem_g1, sem_g2, sem_s, sem_i):
  c = lax.axis_index("c")
  s = lax.axis_index("s")
  w = s * NC + c
  base = w * (CHUNKS * CH)

  zv = jnp.zeros((16,), jnp.float32)
  ov = jnp.ones((16,), jnp.float32)
  for i in range(16):
    for k in range(D // 16):
      zbuf[i, pl.ds(k * 16, 16)] = zv
  for i in range(S // 16):
    cnt[pl.ds(i * 16, 16)] = zv

  sl = pl.ds(s * 16, 16)
  pltpu.sync_copy(zbuf, acc.at[sl])
  plsc.subcore_barrier()

  def count_ids(idx_vals):
    plsc.addupdate_scatter(cnt, [idx_vals], ov)

  bufs = (rows0, rows1, rows2)
  gsems = (sem_g0, sem_g1, sem_g2)

  def gather(b):
    return pltpu.async_copy(feat_hbm.at[pl.ds(base + b * BIG, BIG)],
                            bufs[b % NBUF], gsems[b % NBUF])

  gathers = [gather(0), gather(1)]
  idx_dmas = [
      pltpu.async_copy(seg_hbm.at[pl.ds(base + r * CH, CH)], idx_all.at[r],
                       sem_i)
      for r in range(CHUNKS)
  ]
  for d in idx_dmas:
    d.wait()

  scatters = {}
  for b in range(NBIG):
    for d in scatters.pop(b - 1, ()):
      d.wait()
    if b + 2 < NBIG:
      gathers.append(gather(b + 2))
    gathers[b].wait()
    buf = bufs[b % NBUF]
    scatters[b] = [
        pltpu.async_copy(buf.at[pl.ds(k * CH, CH)],
                         acc.at[idx_all.at[b * (BIG // CH) + k]],
                         sem_s, add=True)
        for k in range(BIG // CH)
    ]
    for k in range(BIG // CH):
      r = b * (BIG // CH) + k
      for q in range(CH // 16):
        count_ids(idx_all[r, pl.ds(q * 16, 16)])
  for ds_ in scatters.values():
    for d in ds_:
      d.wait()

  @pl.when(w < EXTRA)
  def _():
    off = FULL + w * CH
    pltpu.sync_copy(seg_hbm.at[pl.ds(off, CH)], idx_e)
    pltpu.sync_copy(feat_hbm.at[pl.ds(off, CH)], rows0.at[pl.ds(0, CH)])
    pltpu.sync_copy(rows0.at[pl.ds(0, CH)], acc.at[idx_e], add=True)
    for q in range(CH // 16):
      count_ids(idx_e[pl.ds(q * 16, 16)])

  @pl.when(w == EXTRA)
  def _():
    pltpu.sync_copy(seg_hbm.at[pl.ds(TAIL_OFF, TAIL)], idx32)
    pltpu.sync_copy(feat_hbm.at[pl.ds(TAIL_OFF, TAIL)], rows1.at[pl.ds(0, TAIL)])
    pltpu.sync_copy(rows1.at[pl.ds(0, TAIL)], acc.at[idx32], add=True)
    for q in range(TAIL // 16):
      count_ids(idx32[pl.ds(q * 16, 16)])

  plsc.subcore_barrier()

  pltpu.sync_copy(acc.at[sl], psums_hbm.at[c, sl])
  pltpu.sync_copy(cnt, pcnts_hbm.at[w])


_sc_pool = functools.partial(
    pl.kernel,
    out_type=(
        jax.ShapeDtypeStruct((NC, S, D), jnp.float32),
        jax.ShapeDtypeStruct((NW, S), jnp.float32),
    ),
    mesh=plsc.VectorSubcoreMesh(
        core_axis_name="c", subcore_axis_name="s",
        num_cores=NC, num_subcores=NS),
    scratch_types=[
        pltpu.VMEM_SHARED((S, D), jnp.float32),
        pltpu.VMEM((BIG, D), jnp.float32),
        pltpu.VMEM((BIG, D), jnp.float32),
        pltpu.VMEM((BIG, D), jnp.float32),
        pltpu.VMEM((CHUNKS, CH), jnp.int32),
        pltpu.VMEM((CH,), jnp.int32),
        pltpu.VMEM((TAIL,), jnp.int32),
        pltpu.VMEM((S,), jnp.float32),
        pltpu.VMEM((16, D), jnp.float32),
        pltpu.SemaphoreType.DMA,
        pltpu.SemaphoreType.DMA,
        pltpu.SemaphoreType.DMA,
        pltpu.SemaphoreType.DMA,
        pltpu.SemaphoreType.DMA,
    ],
    compiler_params=pltpu.CompilerParams(needs_layout_passes=False),
)(_sc_body)


def _combine_body(ps_ref, pc_ref, o_ref):
  sums = ps_ref[0] + ps_ref[1]
  counts = jnp.sum(pc_ref[...], axis=0, keepdims=True)
  counts_col = jnp.transpose(counts)
  o_ref[...] = sums / jnp.clip(counts_col, 1.0, None)


def kernel(feat, segment_ids, num_graphs):
  seg = segment_ids.astype(jnp.int32)
  psums, pcnts = _sc_pool(feat, seg)
  out = pl.pallas_call(
      _combine_body,
      out_shape=jax.ShapeDtypeStruct((S, D), jnp.float32),
  )(psums, pcnts)
  return out


# speedup vs baseline: 1.1202x; 1.0038x over previous
import functools

import jax
import jax.numpy as jnp
from jax import lax
from jax.experimental import pallas as pl
from jax.experimental.pallas import tpu as pltpu
from jax.experimental.pallas import tpu_sc as plsc

N_ROWS = 100000
D = 128
S = 256
NC = 2
NS = 16
NW = NC * NS
CH = 128
FULL = (N_ROWS // (NW * CH)) * NW * CH
CHUNKS = FULL // (NW * CH)
EXTRA = (N_ROWS - FULL) // CH
TAIL = N_ROWS - FULL - EXTRA * CH
TAIL_OFF = FULL + EXTRA * CH


BIG = 256
NBUF = 3
NBIG = CHUNKS * CH // BIG


def _sc_body(feat_hbm, seg_hbm, psums_hbm, pcnts_hbm,
             acc, rows0, rows1, rows2, rows_e, idx_all, idx_e, idx32, cnt,
             zbuf, sem_g0, sem_g1, sem_g2, sem_s, sem_i, sem_e):
  c = lax.axis_index("c")
  s = lax.axis_index("s")
  w = s * NC + c
  base = w * (CHUNKS * CH)

  zv = jnp.zeros((16,), jnp.float32)
  ov = jnp.ones((16,), jnp.float32)
  for i in range(16):
    for k in range(D // 16):
      zbuf[i, pl.ds(k * 16, 16)] = zv
  for i in range(S // 16):
    cnt[pl.ds(i * 16, 16)] = zv

  sl = pl.ds(s * 16, 16)
  pltpu.sync_copy(zbuf, acc.at[sl])
  plsc.subcore_barrier()

  def count_ids(idx_vals):
    plsc.addupdate_scatter(cnt, [idx_vals], ov)

  bufs = (rows0, rows1, rows2)
  gsems = (sem_g0, sem_g1, sem_g2)

  def gather(b):
    return pltpu.async_copy(feat_hbm.at[pl.ds(base + b * BIG, BIG)],
                            bufs[b % NBUF], gsems[b % NBUF])

  gathers = [gather(0), gather(1)]
  eoff = FULL + w * CH

  @pl.when(w < EXTRA)
  def _():
    pltpu.async_copy(seg_hbm.at[pl.ds(eoff, CH)], idx_e, sem_e)
    pltpu.async_copy(feat_hbm.at[pl.ds(eoff, CH)], rows_e, sem_e)

  @pl.when(w == EXTRA)
  def _():
    pltpu.async_copy(seg_hbm.at[pl.ds(TAIL_OFF, TAIL)], idx32, sem_e)
    pltpu.async_copy(feat_hbm.at[pl.ds(TAIL_OFF, TAIL)],
                     rows_e.at[pl.ds(0, TAIL)], sem_e)
  idx_dmas = [
      pltpu.async_copy(seg_hbm.at[pl.ds(base + r * CH, CH)], idx_all.at[r],
                       sem_i)
      for r in range(CHUNKS)
  ]
  for d in idx_dmas:
    d.wait()

  scatters = {}
  for b in range(NBIG):
    for d in scatters.pop(b - 1, ()):
      d.wait()
    if b + 2 < NBIG:
      gathers.append(gather(b + 2))
    gathers[b].wait()
    buf = bufs[b % NBUF]
    scatters[b] = [
        pltpu.async_copy(buf.at[pl.ds(k * CH, CH)],
                         acc.at[idx_all.at[b * (BIG // CH) + k]],
                         sem_s, add=True)
        for k in range(BIG // CH)
    ]
    for k in range(BIG // CH):
      r = b * (BIG // CH) + k
      for q in range(CH // 16):
        count_ids(idx_all[r, pl.ds(q * 16, 16)])
  for ds_ in scatters.values():
    for d in ds_:
      d.wait()

  @pl.when(w < EXTRA)
  def _():
    pltpu.make_async_copy(seg_hbm.at[pl.ds(eoff, CH)], idx_e, sem_e).wait()
    pltpu.make_async_copy(feat_hbm.at[pl.ds(eoff, CH)], rows_e, sem_e).wait()
    pltpu.sync_copy(rows_e, acc.at[idx_e], add=True)
    for q in range(CH // 16):
      count_ids(idx_e[pl.ds(q * 16, 16)])

  @pl.when(w == EXTRA)
  def _():
    pltpu.make_async_copy(seg_hbm.at[pl.ds(TAIL_OFF, TAIL)], idx32,
                          sem_e).wait()
    pltpu.make_async_copy(feat_hbm.at[pl.ds(TAIL_OFF, TAIL)],
                          rows_e.at[pl.ds(0, TAIL)], sem_e).wait()
    pltpu.sync_copy(rows_e.at[pl.ds(0, TAIL)], acc.at[idx32], add=True)
    for q in range(TAIL // 16):
      count_ids(idx32[pl.ds(q * 16, 16)])

  plsc.subcore_barrier()

  pltpu.sync_copy(acc.at[sl], psums_hbm.at[c, sl])
  pltpu.sync_copy(cnt, pcnts_hbm.at[w])


_sc_pool = functools.partial(
    pl.kernel,
    out_type=(
        jax.ShapeDtypeStruct((NC, S, D), jnp.float32),
        jax.ShapeDtypeStruct((NW, S), jnp.float32),
    ),
    mesh=plsc.VectorSubcoreMesh(
        core_axis_name="c", subcore_axis_name="s",
        num_cores=NC, num_subcores=NS),
    scratch_types=[
        pltpu.VMEM_SHARED((S, D), jnp.float32),
        pltpu.VMEM((BIG, D), jnp.float32),
        pltpu.VMEM((BIG, D), jnp.float32),
        pltpu.VMEM((BIG, D), jnp.float32),
        pltpu.VMEM((CH, D), jnp.float32),
        pltpu.VMEM((CHUNKS, CH), jnp.int32),
        pltpu.VMEM((CH,), jnp.int32),
        pltpu.VMEM((TAIL,), jnp.int32),
        pltpu.VMEM((S,), jnp.float32),
        pltpu.VMEM((16, D), jnp.float32),
        pltpu.SemaphoreType.DMA,
        pltpu.SemaphoreType.DMA,
        pltpu.SemaphoreType.DMA,
        pltpu.SemaphoreType.DMA,
        pltpu.SemaphoreType.DMA,
        pltpu.SemaphoreType.DMA,
    ],
    compiler_params=pltpu.CompilerParams(needs_layout_passes=False),
)(_sc_body)


def _combine_body(ps_ref, pc_ref, o_ref):
  sums = ps_ref[0] + ps_ref[1]
  counts = jnp.sum(pc_ref[...], axis=0, keepdims=True)
  counts_col = jnp.transpose(counts)
  o_ref[...] = sums / jnp.clip(counts_col, 1.0, None)


def kernel(feat, segment_ids, num_graphs):
  seg = segment_ids.astype(jnp.int32)
  psums, pcnts = _sc_pool(feat, seg)
  out = pl.pallas_call(
      _combine_body,
      out_shape=jax.ShapeDtypeStruct((S, D), jnp.float32),
  )(psums, pcnts)
  return out


# speedup vs baseline: 1.1216x; 1.0013x over previous
import functools

import jax
import jax.numpy as jnp
from jax import lax
from jax.experimental import pallas as pl
from jax.experimental.pallas import tpu as pltpu
from jax.experimental.pallas import tpu_sc as plsc

N_ROWS = 100000
D = 128
S = 256
NC = 2
NS = 16
NW = NC * NS
CH = 128
CHUNKS = 18
FULL = NW * CHUNKS * CH
TC_BLK = 512
TC_OFF = FULL // TC_BLK
TC_N = ((N_ROWS - FULL) // TC_BLK) * TC_BLK
EX_OFF = FULL + TC_N
EXTRA = (N_ROWS - EX_OFF) // CH
TAIL = N_ROWS - EX_OFF - EXTRA * CH
TAIL_OFF = EX_OFF + EXTRA * CH


BIG = 256
NBUF = 3
NBIG = CHUNKS * CH // BIG


def _sc_body(feat_hbm, seg_hbm, psums_hbm, pcnts_hbm,
             acc, rows0, rows1, rows2, rows_e, idx_all, idx_e, idx32, cnt,
             zbuf, sem_g0, sem_g1, sem_g2, sem_s, sem_i, sem_e):
  c = lax.axis_index("c")
  s = lax.axis_index("s")
  w = s * NC + c
  base = w * (CHUNKS * CH)

  zv = jnp.zeros((16,), jnp.float32)
  ov = jnp.ones((16,), jnp.float32)
  for i in range(16):
    for k in range(D // 16):
      zbuf[i, pl.ds(k * 16, 16)] = zv
  for i in range(S // 16):
    cnt[pl.ds(i * 16, 16)] = zv

  sl = pl.ds(s * 16, 16)
  pltpu.sync_copy(zbuf, acc.at[sl])
  plsc.subcore_barrier()

  def count_ids(idx_vals):
    plsc.addupdate_scatter(cnt, [idx_vals], ov)

  bufs = (rows0, rows1, rows2)
  gsems = (sem_g0, sem_g1, sem_g2)

  def gather(b):
    return pltpu.async_copy(feat_hbm.at[pl.ds(base + b * BIG, BIG)],
                            bufs[b % NBUF], gsems[b % NBUF])

  gathers = [gather(0), gather(1)]
  eoff = EX_OFF + w * CH

  @pl.when(w < EXTRA)
  def _():
    pltpu.async_copy(seg_hbm.at[pl.ds(eoff, CH)], idx_e, sem_e)
    pltpu.async_copy(feat_hbm.at[pl.ds(eoff, CH)], rows_e, sem_e)

  @pl.when(w == EXTRA)
  def _():
    pltpu.async_copy(seg_hbm.at[pl.ds(TAIL_OFF, TAIL)], idx32, sem_e)
    pltpu.async_copy(feat_hbm.at[pl.ds(TAIL_OFF, TAIL)],
                     rows_e.at[pl.ds(0, TAIL)], sem_e)
  idx_dmas = [
      pltpu.async_copy(seg_hbm.at[pl.ds(base + r * CH, CH)], idx_all.at[r],
                       sem_i)
      for r in range(CHUNKS)
  ]
  for d in idx_dmas:
    d.wait()

  scatters = {}
  for b in range(NBIG):
    for d in scatters.pop(b - 1, ()):
      d.wait()
    if b + 2 < NBIG:
      gathers.append(gather(b + 2))
    gathers[b].wait()
    buf = bufs[b % NBUF]
    scatters[b] = [
        pltpu.async_copy(buf.at[pl.ds(k * CH, CH)],
                         acc.at[idx_all.at[b * (BIG // CH) + k]],
                         sem_s, add=True)
        for k in range(BIG // CH)
    ]
    for k in range(BIG // CH):
      r = b * (BIG // CH) + k
      for q in range(CH // 16):
        count_ids(idx_all[r, pl.ds(q * 16, 16)])
  for ds_ in scatters.values():
    for d in ds_:
      d.wait()

  @pl.when(w < EXTRA)
  def _():
    pltpu.make_async_copy(seg_hbm.at[pl.ds(eoff, CH)], idx_e, sem_e).wait()
    pltpu.make_async_copy(feat_hbm.at[pl.ds(eoff, CH)], rows_e, sem_e).wait()
    pltpu.sync_copy(rows_e, acc.at[idx_e], add=True)
    for q in range(CH // 16):
      count_ids(idx_e[pl.ds(q * 16, 16)])

  @pl.when(w == EXTRA)
  def _():
    pltpu.make_async_copy(seg_hbm.at[pl.ds(TAIL_OFF, TAIL)], idx32,
                          sem_e).wait()
    pltpu.make_async_copy(feat_hbm.at[pl.ds(TAIL_OFF, TAIL)],
                          rows_e.at[pl.ds(0, TAIL)], sem_e).wait()
    pltpu.sync_copy(rows_e.at[pl.ds(0, TAIL)], acc.at[idx32], add=True)
    for q in range(TAIL // 16):
      count_ids(idx32[pl.ds(q * 16, 16)])

  plsc.subcore_barrier()

  pltpu.sync_copy(acc.at[sl], psums_hbm.at[c, sl])
  pltpu.sync_copy(cnt, pcnts_hbm.at[w])


_sc_pool = functools.partial(
    pl.kernel,
    out_type=(
        jax.ShapeDtypeStruct((NC, S, D), jnp.float32),
        jax.ShapeDtypeStruct((NW, S), jnp.float32),
    ),
    mesh=plsc.VectorSubcoreMesh(
        core_axis_name="c", subcore_axis_name="s",
        num_cores=NC, num_subcores=NS),
    scratch_types=[
        pltpu.VMEM_SHARED((S, D), jnp.float32),
        pltpu.VMEM((BIG, D), jnp.float32),
        pltpu.VMEM((BIG, D), jnp.float32),
        pltpu.VMEM((BIG, D), jnp.float32),
        pltpu.VMEM((CH, D), jnp.float32),
        pltpu.VMEM((CHUNKS, CH), jnp.int32),
        pltpu.VMEM((CH,), jnp.int32),
        pltpu.VMEM((TAIL,), jnp.int32),
        pltpu.VMEM((S,), jnp.float32),
        pltpu.VMEM((16, D), jnp.float32),
        pltpu.SemaphoreType.DMA,
        pltpu.SemaphoreType.DMA,
        pltpu.SemaphoreType.DMA,
        pltpu.SemaphoreType.DMA,
        pltpu.SemaphoreType.DMA,
        pltpu.SemaphoreType.DMA,
    ],
    compiler_params=pltpu.CompilerParams(needs_layout_passes=False),
)(_sc_body)


def _tc_body(seg_ref, feat_ref, sums_ref, cnt_ref):
  g = pl.program_id(0)

  @pl.when(g == 0)
  def _():
    sums_ref[...] = jnp.zeros((S, D), jnp.float32)
    cnt_ref[...] = jnp.zeros((S, 1), jnp.float32)

  ids = seg_ref[...]
  seg_col = lax.broadcasted_iota(jnp.int32, (S, 1), 0)
  oh = (ids == seg_col).astype(jnp.float32)
  sums_ref[...] += jnp.dot(oh, feat_ref[...],
                           preferred_element_type=jnp.float32)
  cnt_ref[...] += jnp.sum(oh, axis=1, keepdims=True)


_tc_hist = pl.pallas_call(
    _tc_body,
    grid=(TC_N // TC_BLK,),
    in_specs=[
        pl.BlockSpec((1, TC_BLK), lambda g: (0, g + TC_OFF)),
        pl.BlockSpec((TC_BLK, D), lambda g: (g + TC_OFF, 0)),
    ],
    out_specs=[
        pl.BlockSpec((S, D), lambda g: (0, 0)),
        pl.BlockSpec((S, 1), lambda g: (0, 0)),
    ],
    out_shape=[
        jax.ShapeDtypeStruct((S, D), jnp.float32),
        jax.ShapeDtypeStruct((S, 1), jnp.float32),
    ],
)


def _combine_body(ps_ref, pc_ref, ts_ref, tc_ref, o_ref):
  sums = ps_ref[0] + ps_ref[1] + ts_ref[...]
  counts = jnp.sum(pc_ref[...], axis=0, keepdims=True)
  counts_col = jnp.transpose(counts) + tc_ref[...]
  o_ref[...] = sums / jnp.clip(counts_col, 1.0, None)


def kernel(feat, segment_ids, num_graphs):
  seg = segment_ids.astype(jnp.int32)
  tsums, tcnts = _tc_hist(seg.reshape(1, N_ROWS), feat)
  psums, pcnts = _sc_pool(feat, seg)
  out = pl.pallas_call(
      _combine_body,
      out_shape=jax.ShapeDtypeStruct((S, D), jnp.float32),
  )(psums, pcnts, tsums, tcnts)
  return out
